# baseline (speedup 1.0000x reference)
import jax
import jax.numpy as jnp
import numpy as np
from jax import lax
from jax.experimental import pallas as pl
from jax.experimental.pallas import tpu as pltpu
from jax.experimental.pallas import tpu_sc as plsc

D_MODEL = 256
N_HEADS = 8
N_LEVELS = 4
N_POINTS = 4
HEAD_DIM = D_MODEL // N_HEADS
B = 4
SHAPES_NP = np.array([[64, 64], [32, 32], [16, 16], [8, 8]], dtype=np.int32)
LEN_IN = int((SHAPES_NP[:, 0] * SHAPES_NP[:, 1]).sum())
LEN_Q = LEN_IN
LSI_NP = np.array([0, 4096, 5120, 5376], dtype=np.float32)
NHLP = N_HEADS * N_LEVELS * N_POINTS

NCHUNK = 68
QCHUNK = LEN_Q // NCHUNK
NGROUP = QCHUNK // 16
NCHALF = NCHUNK // 2


def _prep_body(q_ref, ref5_ref, woff_ref, boff_ref, wattn_ref, battn_ref,
               cx_ref, cy_ref, lsi_ref, idx_ref, wt_ref):
    q = q_ref[0]
    soT = lax.dot_general(woff_ref[...], q, (((1,), (1,)), ((), ())),
                          preferred_element_type=jnp.float32)
    soT = soT + boff_ref[...]
    awT = lax.dot_general(wattn_ref[...], q, (((1,), (1,)), ((), ())),
                          preferred_element_type=jnp.float32)
    awT = awT + battn_ref[...]
    parts = []
    for h in range(N_HEADS):
        s = awT[h * 16:(h + 1) * 16, :]
        m = jnp.max(s, axis=0, keepdims=True)
        e = jnp.exp(s - m)
        d = jnp.sum(e, axis=0, keepdims=True)
        parts.append(e / d)
    aw = jnp.concatenate(parts, axis=0)
    refx = ref5_ref[0, 0, 0:1, :]
    refy = ref5_ref[0, 0, 1:2, :]
    cw = cx_ref[...]
    ch = cy_ref[...]
    lsi = lsi_ref[...]
    px = refx * cw + soT[:NHLP] - 0.5
    py = refy * ch + soT[NHLP:] - 0.5
    x0 = jnp.floor(px)
    y0 = jnp.floor(py)
    fx = px - x0
    fy = py - y0
    idxs, wts = [], []
    for c in range(4):
        dx, dy = c % 2, c // 2
        xf = x0 + float(dx)
        yf = y0 + float(dy)
        wx = fx if dx else 1.0 - fx
        wy = fy if dy else 1.0 - fy
        valid = (xf >= 0.0) & (xf <= cw - 1.0) & (yf >= 0.0) & (yf <= ch - 1.0)
        xi = jnp.clip(xf, 0.0, cw - 1.0)
        yi = jnp.clip(yf, 0.0, ch - 1.0)
        rowf = lsi + yi * cw + xi
        idxs.append((rowf * 16.0).astype(jnp.int32))
        wts.append(aw * wx * wy * jnp.where(valid, 1.0, 0.0))
    idx_ref[0, 0] = jnp.concatenate(
        [idxs[c][h * 16:(h + 1) * 16] for h in range(N_HEADS)
         for c in range(4)], axis=0)
    wt_ref[0, 0] = jnp.concatenate(
        [wts[c][h * 16:(h + 1) * 16] for h in range(N_HEADS)
         for c in range(4)], axis=0)


def _tc_prep(query, ref5, W_off2, b_off2, W_attn, b_attn):
    pc = pl.pallas_call(
        _prep_body,
        grid=(B, NCHUNK),
        in_specs=[
            pl.BlockSpec((1, QCHUNK, D_MODEL), lambda b, c: (b, c, 0)),
            pl.BlockSpec((1, 1, 2, QCHUNK), lambda b, c: (b, c, 0, 0)),
            pl.BlockSpec((2 * NHLP, D_MODEL), lambda b, c: (0, 0)),
            pl.BlockSpec((2 * NHLP, 1), lambda b, c: (0, 0)),
            pl.BlockSpec((NHLP, D_MODEL), lambda b, c: (0, 0)),
            pl.BlockSpec((NHLP, 1), lambda b, c: (0, 0)),
            pl.BlockSpec((NHLP, 1), lambda b, c: (0, 0)),
            pl.BlockSpec((NHLP, 1), lambda b, c: (0, 0)),
            pl.BlockSpec((NHLP, 1), lambda b, c: (0, 0)),
        ],
        out_specs=[
            pl.BlockSpec((1, 1, 4 * NHLP, QCHUNK), lambda b, c: (b, c, 0, 0)),
            pl.BlockSpec((1, 1, 4 * NHLP, QCHUNK), lambda b, c: (b, c, 0, 0)),
        ],
        out_shape=[
            jax.ShapeDtypeStruct((B, NCHUNK, 4 * NHLP, QCHUNK), jnp.int32),
            jax.ShapeDtypeStruct((B, NCHUNK, 4 * NHLP, QCHUNK), jnp.float32),
        ],
    )
    cx = jnp.asarray(np.broadcast_to(
        SHAPES_NP[:, 1].astype(np.float32)[None, :, None],
        (N_HEADS, N_LEVELS, N_POINTS)).reshape(NHLP, 1))
    cy = jnp.asarray(np.broadcast_to(
        SHAPES_NP[:, 0].astype(np.float32)[None, :, None],
        (N_HEADS, N_LEVELS, N_POINTS)).reshape(NHLP, 1))
    lsi = jnp.asarray(np.broadcast_to(
        LSI_NP[None, :, None],
        (N_HEADS, N_LEVELS, N_POINTS)).reshape(NHLP, 1))
    return pc(query, ref5, W_off2, b_off2, W_attn, b_attn, cx, cy, lsi)


def _proj_body(x_ref, w_ref, b_ref, o_ref):
    o_ref[0] = lax.dot_general(x_ref[0], w_ref[...], (((1,), (1,)), ((), ())),
                               preferred_element_type=jnp.float32) + b_ref[...]


def _tc_proj(x, W_out, b_out2):
    nblk = 8
    blk = LEN_Q // nblk
    return pl.pallas_call(
        _proj_body,
        grid=(B, nblk),
        in_specs=[
            pl.BlockSpec((1, blk, D_MODEL), lambda b, j: (b, j, 0)),
            pl.BlockSpec((D_MODEL, D_MODEL), lambda b, j: (0, 0)),
            pl.BlockSpec((1, D_MODEL), lambda b, j: (0, 0)),
        ],
        out_specs=pl.BlockSpec((1, blk, D_MODEL), lambda b, j: (b, j, 0)),
        out_shape=jax.ShapeDtypeStruct((B, LEN_Q, D_MODEL), jnp.float32),
    )(x, W_out, b_out2)


def _sc_body(idx_hbm, wt_hbm, feat_hbm, out_hbm,
             feat_v, idx_v, wt_v, out_v, in_sem0, in_sem1, out_sem0, out_sem1):
    cid = lax.axis_index("c")
    sid = lax.axis_index("s")
    wid = sid * 2 + cid
    h = wid % N_HEADS
    rest = wid // N_HEADS
    dh = rest % 2
    qh = rest // 2
    hd = h * 2 + dh

    cols = [jnp.full((16,), d, jnp.int32) for d in range(16)]
    in_sems = (in_sem0, in_sem1)
    out_sems = (out_sem0, out_sem1)

    def in_copies(b, ci, par):
        return (
            pltpu.make_async_copy(
                idx_hbm.at[b, ci, pl.ds(h * 64, 64), :], idx_v.at[par],
                in_sems[par]),
            pltpu.make_async_copy(
                wt_hbm.at[b, ci, pl.ds(h * 64, 64), :], wt_v.at[par],
                in_sems[par]),
        )

    def out_copy(b, ci, par):
        return pltpu.make_async_copy(
            out_v.at[par], out_hbm.at[b, ci, pl.ds(hd * 16, 16), :],
            out_sems[par])

    def body_b(b, _):
        pltpu.sync_copy(feat_hbm.at[b, hd, :], feat_v)
        for par in range(2):
            for cp in in_copies(b, qh * NCHALF + par, par):
                cp.start()

        def body_pair(i, _):
            c2 = i * 2
            for par in range(2):
                c = c2 + par
                ci = qh * NCHALF + c
                for cp in in_copies(b, ci, par):
                    cp.wait()

                @pl.when(c2 > 0)
                def _():
                    out_copy(b, ci - 2, par).wait()

                @plsc.parallel_loop(0, NGROUP)
                def body_g(g):
                    accs = [jnp.zeros((16,), jnp.float32) for _ in range(16)]
                    for r in range(16):
                        for cc in range(4):
                            rowv = idx_v[par, cc * 16 + r, pl.ds(g * 16, 16)]
                            wv = wt_v[par, cc * 16 + r, pl.ds(g * 16, 16)]
                            for dd in range(16):
                                v = plsc.load_gather(feat_v,
                                                     [rowv + cols[dd]])
                                accs[dd] = accs[dd] + wv * v
                    for dd in range(16):
                        out_v[par, dd, pl.ds(g * 16, 16)] = accs[dd]

                out_copy(b, ci, par).start()

                @pl.when(c + 2 < NCHALF)
                def _():
                    for cp in in_copies(b, ci + 2, par):
                        cp.start()

        lax.fori_loop(0, NCHALF // 2, body_pair, None)
        for par in range(2):
            out_copy(b, qh * NCHALF + NCHALF - 2 + par, par).wait()

    lax.fori_loop(0, B, body_b, None)


def _sc_gather(idx5, wt5, feat_t):
    mesh = plsc.VectorSubcoreMesh(core_axis_name="c", subcore_axis_name="s",
                                  num_cores=2, num_subcores=16)
    fn = pl.kernel(
        _sc_body,
        out_type=jax.ShapeDtypeStruct((B, NCHUNK, D_MODEL, QCHUNK),
                                      jnp.float32),
        mesh=mesh,
        compiler_params=pltpu.CompilerParams(use_tc_tiling_on_sc=False,
                                             needs_layout_passes=False,
                                             disable_bounds_checks=True),
        scratch_types=[
            pltpu.VMEM((LEN_IN * 16,), jnp.float32),
            pltpu.VMEM((2, 64, QCHUNK), jnp.int32),
            pltpu.VMEM((2, 64, QCHUNK), jnp.float32),
            pltpu.VMEM((2, 16, QCHUNK), jnp.float32),
            pltpu.SemaphoreType.DMA,
            pltpu.SemaphoreType.DMA,
            pltpu.SemaphoreType.DMA,
            pltpu.SemaphoreType.DMA,
        ],
    )
    return fn(idx5, wt5, feat_t)


def kernel(query, reference_points, input_flatten, spatial_shapes,
           level_start_index, W_off, b_off, W_attn, b_attn, W_out, b_out):
    ref5 = reference_points.transpose(0, 2, 1).reshape(B, 2, NCHUNK, QCHUNK) \
        .transpose(0, 2, 1, 3)
    W_off2 = W_off.reshape(N_HEADS, N_LEVELS, N_POINTS, 2, D_MODEL) \
        .transpose(3, 0, 1, 2, 4).reshape(2 * NHLP, D_MODEL)
    b_off2 = b_off.reshape(N_HEADS, N_LEVELS, N_POINTS, 2) \
        .transpose(3, 0, 1, 2).reshape(2 * NHLP, 1)
    b_attn2 = b_attn.reshape(NHLP, 1)
    feat_t = input_flatten.reshape(B, LEN_IN, 16, 16).transpose(0, 2, 1, 3) \
        .reshape(B, 16, LEN_IN * 16)

    idx5, wt5 = _tc_prep(query, ref5, W_off2, b_off2, W_attn, b_attn2)
    out5 = _sc_gather(idx5, wt5, feat_t)

    x = out5.transpose(0, 1, 3, 2).reshape(B, LEN_Q, D_MODEL)
    return _tc_proj(x, W_out, b_out.reshape(1, D_MODEL))


# speedup vs baseline: 1.6995x; 1.6995x over previous
import jax
import jax.numpy as jnp
import numpy as np
from jax import lax
from jax.experimental import pallas as pl
from jax.experimental.pallas import tpu as pltpu
from jax.experimental.pallas import tpu_sc as plsc

D_MODEL = 256
N_HEADS = 8
N_LEVELS = 4
N_POINTS = 4
HEAD_DIM = D_MODEL // N_HEADS
B = 4
SHAPES_NP = np.array([[64, 64], [32, 32], [16, 16], [8, 8]], dtype=np.int32)
LEN_IN = int((SHAPES_NP[:, 0] * SHAPES_NP[:, 1]).sum())
LEN_Q = LEN_IN
LSI = [0, 4096, 5120, 5376]
NHLP = N_HEADS * N_LEVELS * N_POINTS

NCHUNK = 10
QCHUNK = LEN_Q // NCHUNK
NGROUP = QCHUNK // 16


def _prep_body(q_ref, ref5_ref, woff_ref, boff_ref, wattn_ref, battn_ref,
               cx_ref, cy_ref, px_ref, py_ref, aw_ref):
    q = q_ref[0]
    soT = lax.dot_general(woff_ref[...], q, (((1,), (1,)), ((), ())),
                          preferred_element_type=jnp.float32)
    soT = soT + boff_ref[...]
    awT = lax.dot_general(wattn_ref[...], q, (((1,), (1,)), ((), ())),
                          preferred_element_type=jnp.float32)
    awT = awT + battn_ref[...]
    for h in range(N_HEADS):
        s = awT[h * 16:(h + 1) * 16, :]
        m = jnp.max(s, axis=0, keepdims=True)
        e = jnp.exp(s - m)
        d = jnp.sum(e, axis=0, keepdims=True)
        aw_ref[0, 0, h * 16:(h + 1) * 16, :] = e / d
    refx = ref5_ref[0, 0, 0:1, :]
    refy = ref5_ref[0, 0, 1:2, :]
    px_ref[0, 0] = refx * cx_ref[...] + soT[:NHLP] - 0.5
    py_ref[0, 0] = refy * cy_ref[...] + soT[NHLP:] - 0.5


def _tc_prep(query, ref5, W_off2, b_off2, W_attn, b_attn):
    pc = pl.pallas_call(
        _prep_body,
        grid=(B, NCHUNK),
        in_specs=[
            pl.BlockSpec((1, QCHUNK, D_MODEL), lambda b, c: (b, c, 0)),
            pl.BlockSpec((1, 1, 2, QCHUNK), lambda b, c: (b, c, 0, 0)),
            pl.BlockSpec((2 * NHLP, D_MODEL), lambda b, c: (0, 0)),
            pl.BlockSpec((2 * NHLP, 1), lambda b, c: (0, 0)),
            pl.BlockSpec((NHLP, D_MODEL), lambda b, c: (0, 0)),
            pl.BlockSpec((NHLP, 1), lambda b, c: (0, 0)),
            pl.BlockSpec((NHLP, 1), lambda b, c: (0, 0)),
            pl.BlockSpec((NHLP, 1), lambda b, c: (0, 0)),
        ],
        out_specs=[
            pl.BlockSpec((1, 1, NHLP, QCHUNK), lambda b, c: (b, c, 0, 0)),
            pl.BlockSpec((1, 1, NHLP, QCHUNK), lambda b, c: (b, c, 0, 0)),
            pl.BlockSpec((1, 1, NHLP, QCHUNK), lambda b, c: (b, c, 0, 0)),
        ],
        out_shape=[jax.ShapeDtypeStruct((B, NCHUNK, NHLP, QCHUNK),
                                        jnp.float32)] * 3,
    )
    cx = jnp.asarray(np.broadcast_to(
        SHAPES_NP[:, 1].astype(np.float32)[None, :, None],
        (N_HEADS, N_LEVELS, N_POINTS)).reshape(NHLP, 1))
    cy = jnp.asarray(np.broadcast_to(
        SHAPES_NP[:, 0].astype(np.float32)[None, :, None],
        (N_HEADS, N_LEVELS, N_POINTS)).reshape(NHLP, 1))
    return pc(query, ref5, W_off2, b_off2, W_attn, b_attn, cx, cy)


def _proj_body(x_ref, w_ref, b_ref, o_ref):
    o_ref[0] = lax.dot_general(x_ref[0], w_ref[...], (((1,), (1,)), ((), ())),
                               preferred_element_type=jnp.float32) + b_ref[...]


def _tc_proj(x, W_out, b_out2):
    nblk = 8
    blk = LEN_Q // nblk
    return pl.pallas_call(
        _proj_body,
        grid=(B, nblk),
        in_specs=[
            pl.BlockSpec((1, blk, D_MODEL), lambda b, j: (b, j, 0)),
            pl.BlockSpec((D_MODEL, D_MODEL), lambda b, j: (0, 0)),
            pl.BlockSpec((1, D_MODEL), lambda b, j: (0, 0)),
        ],
        out_specs=pl.BlockSpec((1, blk, D_MODEL), lambda b, j: (b, j, 0)),
        out_shape=jax.ShapeDtypeStruct((B, LEN_Q, D_MODEL), jnp.float32),
    )(x, W_out, b_out2)


def _sc_body(px_hbm, py_hbm, aw_hbm, feat_hbm, out_hbm,
             feat_v, px_v, py_v, aw_v, out_v):
    cid = lax.axis_index("c")
    sid = lax.axis_index("s")
    wid = sid * 2 + cid
    h = wid % N_HEADS
    rest = wid // N_HEADS
    dh = rest % 2
    qh = rest // 2
    hd = h * 2 + dh

    cols = [jnp.full((16,), d, jnp.int32) for d in range(16)]
    one_f = jnp.full((16,), 1.0, jnp.float32)
    zero_f = jnp.full((16,), 0.0, jnp.float32)
    one_i = jnp.full((16,), 1, jnp.int32)
    zero_i = jnp.full((16,), 0, jnp.int32)
    sixteen_i = jnp.full((16,), 16, jnp.int32)

    def body_b(b, _):
        pltpu.sync_copy(feat_hbm.at[b, hd, :], feat_v)

        def body_c(c, _):
            ci = qh * (NCHUNK // 2) + c
            pltpu.sync_copy(px_hbm.at[b, ci, pl.ds(h * 16, 16), :], px_v)
            pltpu.sync_copy(py_hbm.at[b, ci, pl.ds(h * 16, 16), :], py_v)
            pltpu.sync_copy(aw_hbm.at[b, ci, pl.ds(h * 16, 16), :], aw_v)

            @plsc.parallel_loop(0, NGROUP)
            def body_g(g):
                accs = [jnp.zeros((16,), jnp.float32) for _ in range(16)]
                for l in range(N_LEVELS):
                    Wl = int(SHAPES_NP[l, 1])
                    Hl = int(SHAPES_NP[l, 0])
                    base = LSI[l]
                    for p in range(N_POINTS):
                        r = l * N_POINTS + p
                        pxv = px_v[r, pl.ds(g * 16, 16)]
                        pyv = py_v[r, pl.ds(g * 16, 16)]
                        awv = aw_v[r, pl.ds(g * 16, 16)]
                        wmax_i = jnp.full((16,), Wl - 1, jnp.int32)
                        hmax_i = jnp.full((16,), Hl - 1, jnp.int32)
                        base_i = jnp.full((16,), base, jnp.int32)
                        wl_i = jnp.full((16,), Wl, jnp.int32)
                        tx = pxv.astype(jnp.int32)
                        ix0 = tx - jnp.where(tx.astype(jnp.float32) > pxv,
                                             one_i, zero_i)
                        fx = pxv - ix0.astype(jnp.float32)
                        ty = pyv.astype(jnp.int32)
                        iy0 = ty - jnp.where(ty.astype(jnp.float32) > pyv,
                                             one_i, zero_i)
                        fy = pyv - iy0.astype(jnp.float32)
                        ix1 = ix0 + one_i
                        iy1 = iy0 + one_i
                        ex0 = (one_f - fx) * jnp.where(
                            (ix0 >= zero_i) & (ix0 <= wmax_i), one_f, zero_f)
                        ex1 = fx * jnp.where(
                            (ix1 >= zero_i) & (ix1 <= wmax_i), one_f, zero_f)
                        ey0 = (one_f - fy) * jnp.where(
                            (iy0 >= zero_i) & (iy0 <= hmax_i), one_f, zero_f)
                        ey1 = fy * jnp.where(
                            (iy1 >= zero_i) & (iy1 <= hmax_i), one_f, zero_f)
                        xi0 = jnp.minimum(jnp.maximum(ix0, zero_i), wmax_i)
                        xi1 = jnp.minimum(jnp.maximum(ix1, zero_i), wmax_i)
                        yb0 = base_i + jnp.minimum(jnp.maximum(iy0, zero_i),
                                                   hmax_i) * wl_i
                        yb1 = base_i + jnp.minimum(jnp.maximum(iy1, zero_i),
                                                   hmax_i) * wl_i
                        a0 = awv * ey0
                        a1 = awv * ey1
                        corners = ((yb0 + xi0, a0 * ex0), (yb0 + xi1, a0 * ex1),
                                   (yb1 + xi0, a1 * ex0), (yb1 + xi1, a1 * ex1))
                        for rowv, wv in corners:
                            addr = rowv * sixteen_i
                            for dd in range(16):
                                v = plsc.load_gather(feat_v,
                                                     [addr + cols[dd]])
                                accs[dd] = accs[dd] + wv * v
                for dd in range(16):
                    out_v[dd, pl.ds(g * 16, 16)] = accs[dd]

            pltpu.sync_copy(out_v, out_hbm.at[b, ci, pl.ds(hd * 16, 16), :])

        lax.fori_loop(0, NCHUNK // 2, body_c, None)

    lax.fori_loop(0, B, body_b, None)


def _sc_gather(px5, py5, aw5, feat_t):
    mesh = plsc.VectorSubcoreMesh(core_axis_name="c", subcore_axis_name="s",
                                  num_cores=2, num_subcores=16)
    fn = pl.kernel(
        _sc_body,
        out_type=jax.ShapeDtypeStruct((B, NCHUNK, D_MODEL, QCHUNK),
                                      jnp.float32),
        mesh=mesh,
        compiler_params=pltpu.CompilerParams(use_tc_tiling_on_sc=False,
                                             needs_layout_passes=False,
                                             disable_bounds_checks=True),
        scratch_types=[
            pltpu.VMEM((LEN_IN * 16,), jnp.float32),
            pltpu.VMEM((16, QCHUNK), jnp.float32),
            pltpu.VMEM((16, QCHUNK), jnp.float32),
            pltpu.VMEM((16, QCHUNK), jnp.float32),
            pltpu.VMEM((16, QCHUNK), jnp.float32),
        ],
    )
    return fn(px5, py5, aw5, feat_t)


def kernel(query, reference_points, input_flatten, spatial_shapes,
           level_start_index, W_off, b_off, W_attn, b_attn, W_out, b_out):
    ref5 = reference_points.transpose(0, 2, 1).reshape(B, 2, NCHUNK, QCHUNK) \
        .transpose(0, 2, 1, 3)
    W_off2 = W_off.reshape(N_HEADS, N_LEVELS, N_POINTS, 2, D_MODEL) \
        .transpose(3, 0, 1, 2, 4).reshape(2 * NHLP, D_MODEL)
    b_off2 = b_off.reshape(N_HEADS, N_LEVELS, N_POINTS, 2) \
        .transpose(3, 0, 1, 2).reshape(2 * NHLP, 1)
    b_attn2 = b_attn.reshape(NHLP, 1)
    feat_t = input_flatten.reshape(B, LEN_IN, 16, 16).transpose(0, 2, 1, 3) \
        .reshape(B, 16, LEN_IN * 16)

    px5, py5, aw5 = _tc_prep(query, ref5, W_off2, b_off2, W_attn, b_attn2)
    out5 = _sc_gather(px5, py5, aw5, feat_t)

    x = out5.transpose(0, 1, 3, 2).reshape(B, LEN_Q, D_MODEL)
    return _tc_proj(x, W_out, b_out.reshape(1, D_MODEL))
